# baseline (speedup 1.0000x reference)
import jax
import jax.numpy as jnp
from jax import lax
from jax.experimental import pallas as pl
from jax.experimental.pallas import tpu as pltpu
from jax.experimental.pallas import tpu_sc as plsc

NC = 2
NS = 16
NW = NC * NS

LANES = 16
RING = 4
LOOKAHEAD = 2
SLAB = 80


def _compact_call(vocab, embed):
    assert vocab % SLAB == 0 and SLAB % 8 == 0
    n_slabs = vocab // SLAB
    trips = (-(-n_slabs // NW) + RING - 1) // RING * RING
    ev = embed // LANES

    def body(tbl_hbm, out_hbm, b0, b1, b2, b3, f0, f1, f2, f3,
             rs0, rs1, rs2, rs3, ws0, ws1, ws2, ws3):
        bufs = (b0, b1, b2, b3)
        flats = (f0, f1, f2, f3)
        rsem = (rs0, rs1, rs2, rs3)
        wsem = (ws0, ws1, ws2, ws3)
        cid = lax.axis_index("c")
        sid = lax.axis_index("s")
        wid = sid * NC + cid

        def valid(t):
            return t * NW + wid < n_slabs

        def rd(t, b):
            r0 = (t * NW + wid) * SLAB
            return pltpu.make_async_copy(
                tbl_hbm.at[pl.ds(r0, SLAB), :], bufs[b], rsem[b])

        def wr(t, b):
            f0_ = (t * NW + wid) * SLAB * embed
            return pltpu.make_async_copy(
                flats[b], out_hbm.at[pl.ds(f0_, SLAB * embed)], wsem[b])

        rd(0, 0).start()
        rd(1, 1).start()

        def step(i, carry):
            for b in range(RING):
                t = i * RING + b

                @pl.when(valid(t))
                def _(t=t, b=b):
                    rd(t, b).wait()

                    def row(r, _, b=b):
                        for k in range(ev):
                            flats[b][pl.ds(r * embed + k * LANES, LANES)] = (
                                bufs[b][r, pl.ds(k * LANES, LANES)])
                        return 0

                    lax.fori_loop(0, SLAB, row, 0, unroll=8)
                    wr(t, b).start()

                tn = t + LOOKAHEAD
                bn = (b + LOOKAHEAD) % RING

                @pl.when(valid(tn))
                def _(tn=tn, bn=bn):
                    @pl.when(tn >= RING)
                    def _():
                        wr(tn - RING, bn).wait()
                    rd(tn, bn).start()
            return carry

        lax.fori_loop(0, trips // RING, step, 0)

        for t in range(trips - 2 * RING, trips):
            @pl.when(valid(t) & ~valid(t + RING))
            def _(t=t):
                wr(t, t % RING).wait()

    return pl.kernel(
        body,
        out_type=jax.ShapeDtypeStruct((vocab * embed,), jnp.float32),
        mesh=plsc.VectorSubcoreMesh(core_axis_name="c", subcore_axis_name="s"),
        compiler_params=pltpu.CompilerParams(use_tc_tiling_on_sc=True, needs_layout_passes=False),
        scratch_types=[
            pltpu.VMEM((SLAB, embed), jnp.float32),
            pltpu.VMEM((SLAB, embed), jnp.float32),
            pltpu.VMEM((SLAB, embed), jnp.float32),
            pltpu.VMEM((SLAB, embed), jnp.float32),
            pltpu.VMEM((SLAB * embed,), jnp.float32),
            pltpu.VMEM((SLAB * embed,), jnp.float32),
            pltpu.VMEM((SLAB * embed,), jnp.float32),
            pltpu.VMEM((SLAB * embed,), jnp.float32),
            pltpu.SemaphoreType.DMA,
            pltpu.SemaphoreType.DMA,
            pltpu.SemaphoreType.DMA,
            pltpu.SemaphoreType.DMA,
            pltpu.SemaphoreType.DMA,
            pltpu.SemaphoreType.DMA,
            pltpu.SemaphoreType.DMA,
            pltpu.SemaphoreType.DMA,
        ],
    )


def _gather_call(batch, seq, embed, vocab):
    bblk = batch // NW
    e_hi = embed // 8
    ev = embed // LANES

    def body(idx_hbm, tbl_hbm, pe_hbm, out_hbm, idx_v,
             r0, r1, r2, r3, o0, o1, p0, p1, p2, p3,
             g0, g1, g2, g3, s0, s1):
        rows = (r0, r1, r2, r3)
        obs = (o0, o1)
        pes = (p0, p1, p2, p3)
        gsem = (g0, g1, g2, g3)
        ssem = (s0, s1)
        cid = lax.axis_index("c")
        sid = lax.axis_index("s")
        wid = sid * NC + cid

        pltpu.sync_copy(idx_hbm.at[:, pl.ds(wid * bblk, bblk)], idx_v)

        def gather(s, b):
            return pltpu.make_async_copy(
                tbl_hbm.at[idx_v.at[s]], rows[b], gsem[b])

        def perow(s, b):
            return pltpu.make_async_copy(pe_hbm.at[s], pes[b], gsem[b])

        def store(s, b2):
            return pltpu.make_async_copy(
                obs[b2], out_hbm.at[s, :, wid, :, :], ssem[b2])

        gather(0, 0).start()
        perow(0, 0).start()
        gather(1, 1).start()
        perow(1, 1).start()

        lane = lax.broadcasted_iota(jnp.int32, (LANES,), 0)
        tok_idx = [j * LANES + lane for j in range(bblk // LANES)]

        def step(s0_, carry):
            for b in range(RING):
                s = s0_ * RING + b
                b2 = b % 2
                gather(s, b).wait()
                perow(s, b).wait()

                @pl.when(s >= 2)
                def _(s=s, b2=b2):
                    store(s - 2, b2).wait()

                def col(e, _, b=b, b2=b2):
                    evec = jnp.full((LANES,), e, jnp.int32)
                    pe_splat = plsc.load_gather(pes[b], [evec])
                    eh = lax.div(e, 8)
                    el = lax.rem(e, 8)
                    for j in range(bblk // LANES):
                        v = plsc.load_gather(rows[b], [tok_idx[j], evec])
                        obs[b2][eh, el, pl.ds(j * LANES, LANES)] = v + pe_splat
                    return 0

                lax.fori_loop(0, embed, col, 0, unroll=2)
                store(s, b2).start()

                sn = s + LOOKAHEAD
                bn = (b + LOOKAHEAD) % RING

                @pl.when(sn < seq)
                def _(sn=sn, bn=bn):
                    gather(sn, bn).start()
                    perow(sn, bn).start()
            return carry

        lax.fori_loop(0, seq // RING, step, 0)

        store(seq - 2, 0).wait()
        store(seq - 1, 1).wait()

    return pl.kernel(
        body,
        out_type=jax.ShapeDtypeStruct((seq, e_hi, NW, 8, bblk), jnp.float32),
        mesh=plsc.VectorSubcoreMesh(core_axis_name="c", subcore_axis_name="s"),
        compiler_params=pltpu.CompilerParams(use_tc_tiling_on_sc=False, needs_layout_passes=False),
        scratch_types=[
            pltpu.VMEM((seq, bblk), jnp.int32),
            pltpu.VMEM((bblk, embed), jnp.float32),
            pltpu.VMEM((bblk, embed), jnp.float32),
            pltpu.VMEM((bblk, embed), jnp.float32),
            pltpu.VMEM((bblk, embed), jnp.float32),
            pltpu.VMEM((e_hi, 8, bblk), jnp.float32),
            pltpu.VMEM((e_hi, 8, bblk), jnp.float32),
            pltpu.VMEM((embed,), jnp.float32),
            pltpu.VMEM((embed,), jnp.float32),
            pltpu.VMEM((embed,), jnp.float32),
            pltpu.VMEM((embed,), jnp.float32),
            pltpu.SemaphoreType.DMA,
            pltpu.SemaphoreType.DMA,
            pltpu.SemaphoreType.DMA,
            pltpu.SemaphoreType.DMA,
            pltpu.SemaphoreType.DMA,
            pltpu.SemaphoreType.DMA,
        ],
    )


def _pos_encoding(seq_len, d_model):
    pos = jnp.arange(1, 1 + seq_len, dtype=jnp.float32)
    power = jnp.arange(0, d_model, 2, dtype=jnp.float32) / d_model
    divisor = jnp.power(10000.0, power)
    angles = pos[:, None] / divisor[None, :]
    return jnp.stack([jnp.sin(angles), jnp.cos(angles)], axis=-1).reshape(
        seq_len, d_model)


def kernel(inputs, table):
    batch, seq = inputs.shape
    vocab, embed = table.shape
    assert batch % NW == 0

    pe = _pos_encoding(seq, embed)
    tbl_flat = _compact_call(vocab, embed)(table)
    tbl_lin = tbl_flat.reshape(vocab, embed)
    idx_t = inputs.T
    out5 = _gather_call(batch, seq, embed, vocab)(idx_t, tbl_lin, pe)
    bblk = batch // NW
    return out5.transpose(2, 4, 0, 1, 3).reshape(batch, seq, embed)


# speedup vs baseline: 1.0898x; 1.0898x over previous
import jax
import jax.numpy as jnp
from jax import lax
from jax.experimental import pallas as pl
from jax.experimental.pallas import tpu as pltpu
from jax.experimental.pallas import tpu_sc as plsc

NC = 2
NS = 16
NW = NC * NS

LANES = 16
RING = 8
LOOKAHEAD = 4
OB = 2


def _embed_call(batch, seq, embed, vocab):
    bblk = batch // NW
    e_hi = embed // 8
    assert seq % RING == 0 and bblk % LANES == 0

    def body(idx_hbm, tbl_hbm, pe_hbm, out_hbm, idx_v, pe_v,
             r0, r1, r2, r3, r4, r5, r6, r7, o0, o1,
             g0, g1, g2, g3, g4, g5, g6, g7, s0, s1):
        rows = (r0, r1, r2, r3, r4, r5, r6, r7)
        obs = (o0, o1)
        gsem = (g0, g1, g2, g3, g4, g5, g6, g7)
        ssem = (s0, s1)
        cid = lax.axis_index("c")
        sid = lax.axis_index("s")
        wid = sid * NC + cid

        pltpu.sync_copy(idx_hbm.at[:, pl.ds(wid * bblk, bblk)], idx_v)
        pltpu.sync_copy(pe_hbm, pe_v)

        def gather(s, b):
            return pltpu.make_async_copy(
                tbl_hbm.at[idx_v.at[s]], rows[b], gsem[b])

        def store(s, b2):
            return pltpu.make_async_copy(
                obs[b2], out_hbm.at[s, :, wid, :, :], ssem[b2])

        for t in range(LOOKAHEAD):
            gather(t, t).start()

        lane = lax.broadcasted_iota(jnp.int32, (LANES,), 0)
        tok_idx = [j * LANES + lane for j in range(bblk // LANES)]

        def step(s0_, carry):
            for b in range(RING):
                s = s0_ * RING + b
                b2 = b % OB
                gather(s, b).wait()

                @pl.when(s >= OB)
                def _(s=s, b2=b2):
                    store(s - OB, b2).wait()

                svec = jnp.full((LANES,), s, jnp.int32)

                def ehloop(eh, _, b=b, b2=b2, svec=svec):
                    for el in range(8):
                        evec = jnp.full((LANES,), eh * 8 + el, jnp.int32)
                        pe_g = plsc.load_gather(pe_v, [svec, evec])
                        for j in range(bblk // LANES):
                            v = plsc.load_gather(rows[b], [tok_idx[j], evec])
                            obs[b2][eh, el, pl.ds(j * LANES, LANES)] = v + pe_g
                    return 0

                lax.fori_loop(0, e_hi, ehloop, 0)
                store(s, b2).start()

                sn = s + LOOKAHEAD
                bn = (b + LOOKAHEAD) % RING

                @pl.when(sn < seq)
                def _(sn=sn, bn=bn):
                    gather(sn, bn).start()
            return carry

        lax.fori_loop(0, seq // RING, step, 0)

        store(seq - 2, (seq - 2) % OB).wait()
        store(seq - 1, (seq - 1) % OB).wait()

    return pl.kernel(
        body,
        out_type=jax.ShapeDtypeStruct((seq, e_hi, NW, 8, bblk), jnp.float32),
        mesh=plsc.VectorSubcoreMesh(core_axis_name="c", subcore_axis_name="s"),
        compiler_params=pltpu.CompilerParams(
            use_tc_tiling_on_sc=False, needs_layout_passes=False),
        scratch_types=[
            pltpu.VMEM((seq, bblk), jnp.int32),
            pltpu.VMEM((seq, embed), jnp.float32),
            pltpu.VMEM((bblk, embed), jnp.float32),
            pltpu.VMEM((bblk, embed), jnp.float32),
            pltpu.VMEM((bblk, embed), jnp.float32),
            pltpu.VMEM((bblk, embed), jnp.float32),
            pltpu.VMEM((bblk, embed), jnp.float32),
            pltpu.VMEM((bblk, embed), jnp.float32),
            pltpu.VMEM((bblk, embed), jnp.float32),
            pltpu.VMEM((bblk, embed), jnp.float32),
            pltpu.VMEM((e_hi, 8, bblk), jnp.float32),
            pltpu.VMEM((e_hi, 8, bblk), jnp.float32),
            pltpu.SemaphoreType.DMA,
            pltpu.SemaphoreType.DMA,
            pltpu.SemaphoreType.DMA,
            pltpu.SemaphoreType.DMA,
            pltpu.SemaphoreType.DMA,
            pltpu.SemaphoreType.DMA,
            pltpu.SemaphoreType.DMA,
            pltpu.SemaphoreType.DMA,
            pltpu.SemaphoreType.DMA,
            pltpu.SemaphoreType.DMA,
        ],
    )


def _pos_encoding(seq_len, d_model):
    pos = jnp.arange(1, 1 + seq_len, dtype=jnp.float32)
    power = jnp.arange(0, d_model, 2, dtype=jnp.float32) / d_model
    divisor = jnp.power(10000.0, power)
    angles = pos[:, None] / divisor[None, :]
    return jnp.stack([jnp.sin(angles), jnp.cos(angles)], axis=-1).reshape(
        seq_len, d_model)


def kernel(inputs, table):
    batch, seq = inputs.shape
    vocab, embed = table.shape
    assert batch % NW == 0

    pe = _pos_encoding(seq, embed)
    idx_t = inputs.T
    out5 = _embed_call(batch, seq, embed, vocab)(idx_t, table, pe)
    return out5.transpose(2, 4, 0, 1, 3).reshape(batch, seq, embed)


# speedup vs baseline: 1.3773x; 1.2638x over previous
import jax
import jax.numpy as jnp
from jax import lax
from jax.experimental import pallas as pl
from jax.experimental.pallas import tpu as pltpu
from jax.experimental.pallas import tpu_sc as plsc

NC = 2
NS = 16
NW = NC * NS

G1 = 128
RING = 4
LOOKAHEAD = 2


def _make_sc_call(batch, seq, embed, vocab):
    bpw = batch // NW
    assert bpw % RING == 0
    g2 = seq - G1

    def body(idx_hbm, table_hbm, pe_hbm, out_hbm, idx_v, pe_v,
             r0, r1, r2, r3, g0, g1, g2s, g3, s0, s1, s2, s3):
        rows = (r0, r1, r2, r3)
        gsem = (g0, g1, g2s, g3)
        ssem = (s0, s1, s2, s3)
        cid = lax.axis_index("c")
        sid = lax.axis_index("s")
        wid = sid * NC + cid
        base = wid * bpw

        pltpu.sync_copy(idx_hbm.at[pl.ds(base, bpw)], idx_v)
        pltpu.sync_copy(pe_hbm, pe_v)

        def gathers(t, b):
            return (
                pltpu.make_async_copy(
                    table_hbm.at[idx_v.at[t, pl.ds(0, G1)]],
                    rows[b].at[pl.ds(0, G1)], gsem[b]),
                pltpu.make_async_copy(
                    table_hbm.at[idx_v.at[t, pl.ds(G1, g2)]],
                    rows[b].at[pl.ds(G1, g2)], gsem[b]),
            )

        def start_gather(t, b):
            ga, gb = gathers(t, b)
            ga.start()
            gb.start()

        def wait_gather(t, b):
            ga, gb = gathers(t, b)
            ga.wait()
            gb.wait()

        def store(t, b):
            return pltpu.make_async_copy(rows[b], out_hbm.at[base + t], ssem[b])

        start_gather(0, 0)
        start_gather(1, 1)

        nvec = embed // 16

        def outer(t0, carry):
            for b in range(RING):
                t = t0 * RING + b
                wait_gather(t, b)

                def row_body(r, _, b=b):
                    for k in range(nvec):
                        sl = pl.ds(k * 16, 16)
                        rows[b][r, sl] = rows[b][r, sl] + pe_v[r, sl]
                    return 0

                lax.fori_loop(0, seq, row_body, 0, unroll=4)
                store(t, b).start()

                tn = t + LOOKAHEAD
                bn = (b + LOOKAHEAD) % RING

                @pl.when(tn < bpw)
                def _(tn=tn, bn=bn):
                    @pl.when(tn >= RING)
                    def _():
                        store(tn - RING, bn).wait()
                    start_gather(tn, bn)
            return carry

        lax.fori_loop(0, bpw // RING, outer, 0)

        for b in range(RING):
            store(bpw - RING + b, b).wait()

    return pl.kernel(
        body,
        out_type=jax.ShapeDtypeStruct((batch, seq, embed), jnp.float32),
        mesh=plsc.VectorSubcoreMesh(core_axis_name="c", subcore_axis_name="s"),
        compiler_params=pltpu.CompilerParams(use_tc_tiling_on_sc=False),
        scratch_types=[
            pltpu.VMEM((bpw, seq), jnp.int32),
            pltpu.VMEM((seq, embed), jnp.float32),
            pltpu.VMEM((seq, embed), jnp.float32),
            pltpu.VMEM((seq, embed), jnp.float32),
            pltpu.VMEM((seq, embed), jnp.float32),
            pltpu.VMEM((seq, embed), jnp.float32),
            pltpu.SemaphoreType.DMA,
            pltpu.SemaphoreType.DMA,
            pltpu.SemaphoreType.DMA,
            pltpu.SemaphoreType.DMA,
            pltpu.SemaphoreType.DMA,
            pltpu.SemaphoreType.DMA,
            pltpu.SemaphoreType.DMA,
            pltpu.SemaphoreType.DMA,
        ],
    )


def _pos_encoding(seq_len, d_model):
    pos = jnp.arange(1, 1 + seq_len, dtype=jnp.float32)
    power = jnp.arange(0, d_model, 2, dtype=jnp.float32) / d_model
    divisor = jnp.power(10000.0, power)
    angles = pos[:, None] / divisor[None, :]
    return jnp.stack([jnp.sin(angles), jnp.cos(angles)], axis=-1).reshape(
        seq_len, d_model)


def kernel(inputs, table):
    batch, seq = inputs.shape
    vocab, embed = table.shape
    assert batch % NW == 0

    pe = _pos_encoding(seq, embed)
    call = _make_sc_call(batch, seq, embed, vocab)
    return call(inputs, table, pe)


# speedup vs baseline: 1.3799x; 1.0019x over previous
import jax
import jax.numpy as jnp
from jax import lax
from jax.experimental import pallas as pl
from jax.experimental.pallas import tpu as pltpu
from jax.experimental.pallas import tpu_sc as plsc

NC = 2
NS = 16
NW = NC * NS

G1 = 128
RING = 4
LOOKAHEAD = 3


def _make_sc_call(batch, seq, embed, vocab):
    bpw = batch // NW
    assert bpw % RING == 0
    g2 = seq - G1

    def body(idx_hbm, table_hbm, pe_hbm, out_hbm, idx_v, pe_v,
             r0, r1, r2, r3, g0, g1, g2s, g3, s0, s1, s2, s3):
        rows = (r0, r1, r2, r3)
        gsem = (g0, g1, g2s, g3)
        ssem = (s0, s1, s2, s3)
        cid = lax.axis_index("c")
        sid = lax.axis_index("s")
        wid = sid * NC + cid
        base = wid * bpw

        pltpu.sync_copy(idx_hbm.at[pl.ds(base, bpw)], idx_v)
        pltpu.sync_copy(pe_hbm, pe_v)

        def gathers(t, b):
            return (
                pltpu.make_async_copy(
                    table_hbm.at[idx_v.at[t, pl.ds(0, G1)]],
                    rows[b].at[pl.ds(0, G1)], gsem[b]),
                pltpu.make_async_copy(
                    table_hbm.at[idx_v.at[t, pl.ds(G1, g2)]],
                    rows[b].at[pl.ds(G1, g2)], gsem[b]),
            )

        def start_gather(t, b):
            ga, gb = gathers(t, b)
            ga.start()
            gb.start()

        def wait_gather(t, b):
            ga, gb = gathers(t, b)
            ga.wait()
            gb.wait()

        def store(t, b):
            return pltpu.make_async_copy(rows[b], out_hbm.at[base + t], ssem[b])

        for t in range(LOOKAHEAD):
            start_gather(t, t)

        nvec = embed // 16

        def outer(t0, carry):
            for b in range(RING):
                t = t0 * RING + b
                wait_gather(t, b)

                def row_body(r, _, b=b):
                    for k in range(nvec):
                        sl = pl.ds(k * 16, 16)
                        rows[b][r, sl] = rows[b][r, sl] + pe_v[r, sl]
                    return 0

                lax.fori_loop(0, seq, row_body, 0, unroll=4)
                store(t, b).start()

                tn = t + LOOKAHEAD
                bn = (b + LOOKAHEAD) % RING

                @pl.when(tn < bpw)
                def _(tn=tn, bn=bn):
                    @pl.when(tn >= RING)
                    def _():
                        store(tn - RING, bn).wait()
                    start_gather(tn, bn)
            return carry

        lax.fori_loop(0, bpw // RING, outer, 0)

        for b in range(RING):
            store(bpw - RING + b, b).wait()

    return pl.kernel(
        body,
        out_type=jax.ShapeDtypeStruct((batch, seq, embed), jnp.float32),
        mesh=plsc.VectorSubcoreMesh(core_axis_name="c", subcore_axis_name="s"),
        compiler_params=pltpu.CompilerParams(use_tc_tiling_on_sc=False),
        scratch_types=[
            pltpu.VMEM((bpw, seq), jnp.int32),
            pltpu.VMEM((seq, embed), jnp.float32),
            pltpu.VMEM((seq, embed), jnp.float32),
            pltpu.VMEM((seq, embed), jnp.float32),
            pltpu.VMEM((seq, embed), jnp.float32),
            pltpu.VMEM((seq, embed), jnp.float32),
            pltpu.SemaphoreType.DMA,
            pltpu.SemaphoreType.DMA,
            pltpu.SemaphoreType.DMA,
            pltpu.SemaphoreType.DMA,
            pltpu.SemaphoreType.DMA,
            pltpu.SemaphoreType.DMA,
            pltpu.SemaphoreType.DMA,
            pltpu.SemaphoreType.DMA,
        ],
    )


def _pos_encoding(seq_len, d_model):
    pos = jnp.arange(1, 1 + seq_len, dtype=jnp.float32)
    power = jnp.arange(0, d_model, 2, dtype=jnp.float32) / d_model
    divisor = jnp.power(10000.0, power)
    angles = pos[:, None] / divisor[None, :]
    return jnp.stack([jnp.sin(angles), jnp.cos(angles)], axis=-1).reshape(
        seq_len, d_model)


def kernel(inputs, table):
    batch, seq = inputs.shape
    vocab, embed = table.shape
    assert batch % NW == 0

    pe = _pos_encoding(seq, embed)
    call = _make_sc_call(batch, seq, embed, vocab)
    return call(inputs, table, pe)
